# Initial kernel scaffold; baseline (speedup 1.0000x reference)
import functools

import jax
import jax.numpy as jnp
from jax import lax
from jax.experimental import pallas as pl
from jax.experimental.pallas import tpu as pltpu
from jax.experimental.pallas import tpu_sc as plsc

B = 4096
L = 50
DIM = 64
NC = 2
NS = 16
NW = NC * NS
SPW = B // NW
CH = 16
NCHUNK = SPW // CH
TOK = CH * L
LANES = 16
NQ = DIM // LANES

_SUBS = [(o, 128) for o in range(0, TOK - 128 + 1, 128)]
if TOK % 128:
    _SUBS.append((TOK - TOK % 128, TOK % 128))


def _pool_kernel(xf_hbm, len_hbm, table_hbm, rep_hbm,
                 idx_buf, len_buf, rows_buf, rep_buf, sem):
    wid = lax.axis_index("s") * NC + lax.axis_index("c")

    def chunk_body(c, carry):
        base = wid * SPW + c * CH
        tok = base * L
        pltpu.sync_copy(xf_hbm.at[pl.ds(tok, TOK)], idx_buf)
        pltpu.sync_copy(len_hbm.at[pl.ds(base, CH)], len_buf)

        copies = [
            pltpu.make_async_copy(
                table_hbm.at[idx_buf.at[pl.ds(o, n)]],
                rows_buf.at[pl.ds(o, n)],
                sem,
            )
            for (o, n) in _SUBS
        ]
        for cp in copies:
            cp.start()
        for cp in copies:
            cp.wait()

        lenv = len_buf[...]
        lenf = lenv.astype(jnp.float32)
        invv = 1.0 / lenf

        for s in range(CH):
            n_s = lenv[s]

            def tok_body(j, accs, s=s):
                row = s * L + j
                return tuple(
                    accs[q] + rows_buf[row, pl.ds(q * LANES, LANES)]
                    for q in range(NQ)
                )

            accs = lax.fori_loop(
                0, n_s, tok_body,
                tuple(jnp.zeros((LANES,), jnp.float32) for _ in range(NQ)),
            )
            inv_s = invv[s]
            for q in range(NQ):
                rep_buf[s, pl.ds(q * LANES, LANES)] = accs[q] * inv_s

        pltpu.sync_copy(rep_buf, rep_hbm.at[pl.ds(base, CH)])
        return carry

    lax.fori_loop(0, NCHUNK, chunk_body, 0)


@functools.partial(
    pl.kernel,
    out_type=jax.ShapeDtypeStruct((B, DIM), jnp.float32),
    mesh=plsc.VectorSubcoreMesh(core_axis_name="c", subcore_axis_name="s"),
    scratch_types=[
        pltpu.VMEM((TOK,), jnp.int32),
        pltpu.VMEM((CH,), jnp.int32),
        pltpu.VMEM((TOK, DIM), jnp.float32),
        pltpu.VMEM((CH, DIM), jnp.float32),
        pltpu.SemaphoreType.DMA,
    ],
)
def _pool(xf_hbm, len_hbm, table_hbm, rep_hbm,
          idx_buf, len_buf, rows_buf, rep_buf, sem):
    _pool_kernel(xf_hbm, len_hbm, table_hbm, rep_hbm,
                 idx_buf, len_buf, rows_buf, rep_buf, sem)


def _mlp_body(rep_ref, w1_ref, b1_ref, w2_ref, b2_ref, out_ref):
    rep = rep_ref[...]
    h = lax.dot_general(rep, w1_ref[...], (((1,), (1,)), ((), ())),
                        preferred_element_type=jnp.float32)
    h = jnp.maximum(h + b1_ref[...], 0.0)
    out = lax.dot_general(h, w2_ref[...], (((1,), (1,)), ((), ())),
                          preferred_element_type=jnp.float32)
    out_ref[...] = out + b2_ref[...]


def _mlp(rep, W1, b1, W2, b2):
    return pl.pallas_call(
        _mlp_body,
        out_shape=jax.ShapeDtypeStruct((B, W2.shape[0]), jnp.float32),
    )(rep, W1, b1.reshape(1, -1), W2, b2.reshape(1, -1))


@jax.jit
def kernel(x, lengths, table, W1, b1, W2, b2):
    xf = x.reshape(B * L).astype(jnp.int32)
    lens = lengths.astype(jnp.int32)
    rep = _pool(xf, lens, table)
    return _mlp(rep, W1, b1, W2, b2)


# speedup vs baseline: 7.9100x; 7.9100x over previous
import functools

import jax
import jax.numpy as jnp
from jax import lax
from jax.experimental import pallas as pl
from jax.experimental.pallas import tpu as pltpu
from jax.experimental.pallas import tpu_sc as plsc

B = 4096
L = 50
DIM = 64
NC = 2
NS = 16
NW = NC * NS
SPW = B // NW
CH = 16
NCHUNK = SPW // CH
TOK = CH * L
LANES = 16
NQ = DIM // LANES

_SUBS = [(o, 128) for o in range(0, TOK - 128 + 1, 128)]
if TOK % 128:
    _SUBS.append((TOK - TOK % 128, TOK % 128))


def _pool_kernel(xf_hbm, len_hbm, table_hbm, rep_hbm,
                 idx_buf, len_buf, rows_buf, rep_buf, sem):
    wid = lax.axis_index("s") * NC + lax.axis_index("c")

    def chunk_body(c, carry):
        base = wid * SPW + c * CH
        tok = base * L
        pltpu.sync_copy(xf_hbm.at[pl.ds(tok, TOK)], idx_buf)
        pltpu.sync_copy(len_hbm.at[pl.ds(base, CH)], len_buf)

        copies = [
            pltpu.make_async_copy(
                table_hbm.at[idx_buf.at[pl.ds(o, n)]],
                rows_buf.at[pl.ds(o, n)],
                sem,
            )
            for (o, n) in _SUBS
        ]
        for cp in copies:
            cp.start()
        for cp in copies:
            cp.wait()

        lenv = len_buf[...]
        lenf = lenv.astype(jnp.float32)
        invv = 1.0 / lenf

        for s in range(CH):
            n_s = lenv[s]

            def tok_body(j, accs, s=s):
                row = s * L + j
                return tuple(
                    accs[q] + rows_buf[row, pl.ds(q * LANES, LANES)]
                    for q in range(NQ)
                )

            accs = lax.fori_loop(
                0, n_s, tok_body,
                tuple(jnp.zeros((LANES,), jnp.float32) for _ in range(NQ)),
            )
            inv_s = invv[s]
            for q in range(NQ):
                rep_buf[s, pl.ds(q * LANES, LANES)] = accs[q] * inv_s

        pltpu.sync_copy(rep_buf, rep_hbm.at[pl.ds(base, CH)])
        return carry

    lax.fori_loop(0, NCHUNK, chunk_body, 0)


@functools.lru_cache(maxsize=1)
def _make_pool():
    return pl.kernel(
        _pool_kernel,
        out_type=jax.ShapeDtypeStruct((B, DIM), jnp.float32),
        mesh=plsc.VectorSubcoreMesh(core_axis_name="c", subcore_axis_name="s",
                                    num_cores=NC, num_subcores=NS),
        compiler_params=pltpu.CompilerParams(use_tc_tiling_on_sc=False),
        scratch_types=[
            pltpu.VMEM((TOK,), jnp.int32),
            pltpu.VMEM((CH,), jnp.int32),
            pltpu.VMEM((TOK, DIM), jnp.float32),
            pltpu.VMEM((CH, DIM), jnp.float32),
            pltpu.SemaphoreType.DMA,
        ],
    )


def _mlp_body(rep_ref, w1_ref, b1_ref, w2_ref, b2_ref, out_ref):
    rep = rep_ref[...]
    h = lax.dot_general(rep, w1_ref[...], (((1,), (1,)), ((), ())),
                        preferred_element_type=jnp.float32)
    h = jnp.maximum(h + b1_ref[...], 0.0)
    out = lax.dot_general(h, w2_ref[...], (((1,), (1,)), ((), ())),
                          preferred_element_type=jnp.float32)
    out_ref[...] = out + b2_ref[...]


def _mlp(rep, W1, b1, W2, b2):
    return pl.pallas_call(
        _mlp_body,
        out_shape=jax.ShapeDtypeStruct((B, W2.shape[0]), jnp.float32),
    )(rep, W1, b1.reshape(1, -1), W2, b2.reshape(1, -1))


@jax.jit
def kernel(x, lengths, table, W1, b1, W2, b2):
    xf = x.reshape(B * L).astype(jnp.int32)
    lens = lengths.astype(jnp.int32)
    rep = _make_pool()(xf, lens, table)
    return _mlp(rep, W1, b1, W2, b2)


# speedup vs baseline: 8.9667x; 1.1336x over previous
import functools

import jax
import jax.numpy as jnp
from jax import lax
from jax.experimental import pallas as pl
from jax.experimental.pallas import tpu as pltpu
from jax.experimental.pallas import tpu_sc as plsc

B = 4096
L = 50
DIM = 64
NC = 2
NS = 16
NW = NC * NS
SPW = B // NW
CH = 16
NCHUNK = SPW // CH
TOK = CH * L
LANES = 16
NQ = DIM // LANES

_SUBS = [(o, 128) for o in range(0, TOK - 128 + 1, 128)]
if TOK % 128:
    _SUBS.append((TOK - TOK % 128, TOK % 128))


def _pool_kernel(xf_hbm, len_hbm, table_hbm, rep_hbm,
                 idx_buf, len_buf, rows_buf, rep_buf, sem0, sem1):
    wid = lax.axis_index("s") * NC + lax.axis_index("c")
    sems = (sem0, sem1)

    def gathers(b, sem):
        return [
            pltpu.make_async_copy(
                table_hbm.at[idx_buf.at[b].at[pl.ds(o, n)]],
                rows_buf.at[b].at[pl.ds(o, n)],
                sem,
            )
            for (o, n) in _SUBS
        ]

    def start(c, b):
        base = wid * SPW + c * CH
        pltpu.sync_copy(xf_hbm.at[pl.ds(base * L, TOK)], idx_buf.at[b])
        pltpu.sync_copy(len_hbm.at[pl.ds(base, CH)], len_buf.at[b])
        for cp in gathers(b, sems[b]):
            cp.start()

    def finish(c, b):
        for cp in gathers(b, sems[b]):
            cp.wait()

        lenv = len_buf[b, :]
        lenf = lenv.astype(jnp.float32)
        invv = 1.0 / lenf

        for s in range(CH):
            n_s = lenv[s]

            def tok_body(j, accs, s=s):
                row = s * L + j
                return tuple(
                    accs[q] + rows_buf[b, row, pl.ds(q * LANES, LANES)]
                    for q in range(NQ)
                )

            accs = lax.fori_loop(
                0, n_s, tok_body,
                tuple(jnp.zeros((LANES,), jnp.float32) for _ in range(NQ)),
            )
            inv_s = invv[s]
            for q in range(NQ):
                rep_buf[s, pl.ds(q * LANES, LANES)] = accs[q] * inv_s

        base = wid * SPW + c * CH
        pltpu.sync_copy(rep_buf, rep_hbm.at[pl.ds(base, CH)])

    start(0, 0)

    def pair_body(p, carry):
        c0 = 2 * p
        start(c0 + 1, 1)
        finish(c0, 0)

        @pl.when(p < NCHUNK // 2 - 1)
        def _():
            start(c0 + 2, 0)

        finish(c0 + 1, 1)
        return carry

    lax.fori_loop(0, NCHUNK // 2, pair_body, 0)


@functools.lru_cache(maxsize=1)
def _make_pool():
    return pl.kernel(
        _pool_kernel,
        out_type=jax.ShapeDtypeStruct((B, DIM), jnp.float32),
        mesh=plsc.VectorSubcoreMesh(core_axis_name="c", subcore_axis_name="s",
                                    num_cores=NC, num_subcores=NS),
        compiler_params=pltpu.CompilerParams(use_tc_tiling_on_sc=False),
        scratch_types=[
            pltpu.VMEM((2, TOK), jnp.int32),
            pltpu.VMEM((2, CH), jnp.int32),
            pltpu.VMEM((2, TOK, DIM), jnp.float32),
            pltpu.VMEM((CH, DIM), jnp.float32),
            pltpu.SemaphoreType.DMA,
            pltpu.SemaphoreType.DMA,
        ],
    )


def _mlp_body(rep_ref, w1_ref, b1_ref, w2_ref, b2_ref, out_ref):
    rep = rep_ref[...]
    h = lax.dot_general(rep, w1_ref[...], (((1,), (1,)), ((), ())),
                        preferred_element_type=jnp.float32)
    h = jnp.maximum(h + b1_ref[...], 0.0)
    out = lax.dot_general(h, w2_ref[...], (((1,), (1,)), ((), ())),
                          preferred_element_type=jnp.float32)
    out_ref[...] = out + b2_ref[...]


def _mlp(rep, W1, b1, W2, b2):
    return pl.pallas_call(
        _mlp_body,
        out_shape=jax.ShapeDtypeStruct((B, W2.shape[0]), jnp.float32),
    )(rep, W1, b1.reshape(1, -1), W2, b2.reshape(1, -1))


@jax.jit
def kernel(x, lengths, table, W1, b1, W2, b2):
    xf = x.reshape(B * L).astype(jnp.int32)
    lens = lengths.astype(jnp.int32)
    rep = _make_pool()(xf, lens, table)
    return _mlp(rep, W1, b1, W2, b2)


# speedup vs baseline: 9.1437x; 1.0197x over previous
import functools

import jax
import jax.numpy as jnp
from jax import lax
from jax.experimental import pallas as pl
from jax.experimental.pallas import tpu as pltpu
from jax.experimental.pallas import tpu_sc as plsc

B = 4096
L = 50
DIM = 64
NC = 2
NS = 16
NW = NC * NS
SPW = B // NW
CH = 16
NCHUNK = SPW // CH
TOK = CH * L
LANES = 16
NQ = DIM // LANES
LX = 64
LG = 50

def _pool_kernel(x_hbm, len_hbm, table_hbm, rep_hbm,
                 idx_buf, len_buf, rows_buf, rep_buf, sem0, sem1):
    wid = lax.axis_index("s") * NC + lax.axis_index("c")
    sems = (sem0, sem1)

    def gathers(b, sem):
        return [
            pltpu.make_async_copy(
                table_hbm.at[idx_buf.at[b].at[pl.ds(s * LX, LG)]],
                rows_buf.at[b].at[pl.ds(s * LG, LG)],
                sem,
            )
            for s in range(CH)
        ]

    def start(c, b):
        base = wid * SPW + c * CH
        pltpu.sync_copy(x_hbm.at[pl.ds(base * LX, CH * LX)], idx_buf.at[b])
        pltpu.sync_copy(len_hbm.at[pl.ds(base, CH)], len_buf.at[b])
        for cp in gathers(b, sems[b]):
            cp.start()

    def finish(c, b):
        for cp in gathers(b, sems[b]):
            cp.wait()

        lenv = len_buf[b, :]
        lenf = lenv.astype(jnp.float32)
        invv = 1.0 / lenf

        for s in range(CH):
            n_s = lenv[s]
            rem = lax.rem(n_s, 2)
            n2 = lax.div(n_s, 2)
            remf = rem.astype(jnp.float32)

            accs = tuple(
                rows_buf[b, s * LG, pl.ds(q * LANES, LANES)] * remf
                for q in range(NQ)
            )

            def tok_body(j, accs, s=s, rem=rem):
                row = s * LG + rem + 2 * j
                return tuple(
                    accs[q]
                    + rows_buf[b, row, pl.ds(q * LANES, LANES)]
                    + rows_buf[b, row + 1, pl.ds(q * LANES, LANES)]
                    for q in range(NQ)
                )

            accs = lax.fori_loop(0, n2, tok_body, accs)
            inv_s = invv[s]
            for q in range(NQ):
                rep_buf[s, pl.ds(q * LANES, LANES)] = accs[q] * inv_s

        base = wid * SPW + c * CH
        pltpu.sync_copy(rep_buf, rep_hbm.at[pl.ds(base, CH)])

    start(0, 0)

    def pair_body(p, carry):
        c0 = 2 * p
        start(c0 + 1, 1)
        finish(c0, 0)

        @pl.when(p < NCHUNK // 2 - 1)
        def _():
            start(c0 + 2, 0)

        finish(c0 + 1, 1)
        return carry

    lax.fori_loop(0, NCHUNK // 2, pair_body, 0)


@functools.lru_cache(maxsize=1)
def _make_pool():
    return pl.kernel(
        _pool_kernel,
        out_type=jax.ShapeDtypeStruct((B, DIM), jnp.float32),
        mesh=plsc.VectorSubcoreMesh(core_axis_name="c", subcore_axis_name="s",
                                    num_cores=NC, num_subcores=NS),
        compiler_params=pltpu.CompilerParams(use_tc_tiling_on_sc=False),
        scratch_types=[
            pltpu.VMEM((2, CH * LX), jnp.int32),
            pltpu.VMEM((2, CH), jnp.int32),
            pltpu.VMEM((2, CH * LG, DIM), jnp.float32),
            pltpu.VMEM((CH, DIM), jnp.float32),
            pltpu.SemaphoreType.DMA,
            pltpu.SemaphoreType.DMA,
        ],
    )


def _mlp_body(rep_ref, w1_ref, b1_ref, w2_ref, b2_ref, out_ref):
    rep = rep_ref[...]
    h = lax.dot_general(rep, w1_ref[...], (((1,), (1,)), ((), ())),
                        preferred_element_type=jnp.float32)
    h = jnp.maximum(h + b1_ref[...], 0.0)
    out = lax.dot_general(h, w2_ref[...], (((1,), (1,)), ((), ())),
                          preferred_element_type=jnp.float32)
    out_ref[...] = out + b2_ref[...]


def _mlp(rep, W1, b1, W2, b2):
    return pl.pallas_call(
        _mlp_body,
        out_shape=jax.ShapeDtypeStruct((B, W2.shape[0]), jnp.float32),
    )(rep, W1, b1.reshape(1, -1), W2, b2.reshape(1, -1))


@jax.jit
def kernel(x, lengths, table, W1, b1, W2, b2):
    padv = jnp.broadcast_to(
        (jnp.arange(LX - L, dtype=jnp.int32) * 4093)[None, :], (B, LX - L)
    )
    xi = jnp.concatenate([x.astype(jnp.int32), padv], axis=1).reshape(B * LX)
    lens = lengths.astype(jnp.int32)
    rep = _make_pool()(xi, lens, table)
    return _mlp(rep, W1, b1, W2, b2)
